# Initial kernel scaffold; baseline (speedup 1.0000x reference)
import jax
import jax.numpy as jnp
from jax.experimental import pallas as pl


def kernel(query, reference_points, value, Wv, bv, Wo, bo, Wa, ba, Wout, bout, spatial_shape):
    raise NotImplementedError("write your pallas kernel here")


# speedup vs baseline: 5.0551x; 5.0551x over previous
import functools
import math

import jax
import jax.numpy as jnp
from jax import lax
from jax.experimental import pallas as pl
from jax.experimental.pallas import tpu as pltpu
from jax.experimental.pallas import tpu_sc as plsc

EMBED = 256
NH = 8
NP = 4
HD = EMBED // NH
NCORN = 4
NS_PER = NP * NCORN
NC, NS, LANES = 2, 16, 16
NW = NC * NS


def _matmul_bias_body(x_ref, w_ref, b_ref, o_ref):
    o_ref[...] = (
        jnp.dot(x_ref[...], w_ref[...], preferred_element_type=jnp.float32)
        + b_ref[...]
    )


def _matmul_bias(x, w, b, blk):
    m, k = x.shape
    n = w.shape[1]
    return pl.pallas_call(
        _matmul_bias_body,
        grid=(m // blk,),
        in_specs=[
            pl.BlockSpec((blk, k), lambda i: (i, 0)),
            pl.BlockSpec((k, n), lambda i: (0, 0)),
            pl.BlockSpec((1, n), lambda i: (0, 0)),
        ],
        out_specs=pl.BlockSpec((blk, n), lambda i: (i, 0)),
        out_shape=jax.ShapeDtypeStruct((m, n), jnp.float32),
    )(x, w, b)


def _prep_body(q_ref, rp_ref, wox_ref, woy_ref, wa_ref, box_ref, boy_ref,
               ba_ref, idx_ref, wgt_ref, *, Hs, Ws, HW, TQ):
    b = pl.program_id(0)
    q = q_ref[0]
    rp = rp_ref[0]
    rpx = rp[:, 0:1]
    rpy = rp[:, 1:2]
    base = b * (HW * NH)
    for h in range(NH):
        offx = jnp.dot(q, wox_ref[h], preferred_element_type=jnp.float32) + box_ref[h]
        offy = jnp.dot(q, woy_ref[h], preferred_element_type=jnp.float32) + boy_ref[h]
        lg = jnp.dot(q, wa_ref[h], preferred_element_type=jnp.float32) + ba_ref[h]
        m = jnp.max(lg, axis=1, keepdims=True)
        e = jnp.exp(lg - m)
        aw = e / jnp.sum(e, axis=1, keepdims=True)
        x = rpx * Ws + offx - 0.5
        y = rpy * Hs + offy - 0.5
        x0 = jnp.floor(x)
        fx = x - x0
        y0 = jnp.floor(y)
        fy = y - y0
        idx_parts = []
        wgt_parts = []
        for dy in (0, 1):
            for dx in (0, 1):
                ix = x0 + dx
                iy = y0 + dy
                valid = ((ix >= 0) & (ix <= Ws - 1)
                         & (iy >= 0) & (iy <= Hs - 1))
                wc = (fx if dx else 1.0 - fx) * (fy if dy else 1.0 - fy)
                wgt_parts.append(jnp.where(valid, aw * wc, 0.0))
                ixc = jnp.clip(ix, 0, Ws - 1).astype(jnp.int32)
                iyc = jnp.clip(iy, 0, Hs - 1).astype(jnp.int32)
                idx_parts.append(base + (iyc * Ws + ixc) * NH + h)
        idx_ref[0, h] = jnp.concatenate(idx_parts, axis=1)
        wgt_ref[0, h] = jnp.concatenate(wgt_parts, axis=1)
    return


def _sampling_prep(query, rp, wox, woy, wa, box, boy, ba3, Hs, Ws, TQ):
    bs, nq, _ = query.shape
    HW = Hs * Ws
    body = functools.partial(_prep_body, Hs=Hs, Ws=Ws, HW=HW, TQ=TQ)
    return pl.pallas_call(
        body,
        grid=(bs, nq // TQ),
        in_specs=[
            pl.BlockSpec((1, TQ, EMBED), lambda b, t: (b, t, 0)),
            pl.BlockSpec((1, TQ, 2), lambda b, t: (b, t, 0)),
            pl.BlockSpec((NH, EMBED, NP), lambda b, t: (0, 0, 0)),
            pl.BlockSpec((NH, EMBED, NP), lambda b, t: (0, 0, 0)),
            pl.BlockSpec((NH, EMBED, NP), lambda b, t: (0, 0, 0)),
            pl.BlockSpec((NH, 1, NP), lambda b, t: (0, 0, 0)),
            pl.BlockSpec((NH, 1, NP), lambda b, t: (0, 0, 0)),
            pl.BlockSpec((NH, 1, NP), lambda b, t: (0, 0, 0)),
        ],
        out_specs=[
            pl.BlockSpec((1, NH, TQ, NS_PER), lambda b, t: (b, 0, t, 0)),
            pl.BlockSpec((1, NH, TQ, NS_PER), lambda b, t: (b, 0, t, 0)),
        ],
        out_shape=[
            jax.ShapeDtypeStruct((bs, NH, nq, NS_PER), jnp.int32),
            jax.ShapeDtypeStruct((bs, NH, nq, NS_PER), jnp.float32),
        ],
    )(query, rp, wox, woy, wa, box, boy, ba3)


def _bcast_lane(vec, s):
    return jnp.broadcast_to(lax.slice(vec, (s,), (s + 1,)), (LANES,))


def _sc_sample_combine(table, idx2d, wgt3, bs, nq):
    C = 128
    NCH = nq // C
    GR = C * NS_PER // 128
    mesh = plsc.VectorSubcoreMesh(core_axis_name="c", subcore_axis_name="s")

    @functools.partial(
        pl.kernel,
        mesh=mesh,
        compiler_params=pltpu.CompilerParams(use_tc_tiling_on_sc=False),
        out_type=jax.ShapeDtypeStruct((bs, NH, nq, HD), jnp.float32),
        scratch_types=[
            pltpu.VMEM((GR, 128), jnp.int32),
            pltpu.VMEM((C, NS_PER), jnp.float32),
            pltpu.VMEM((C * NS_PER, HD), jnp.float32),
            pltpu.VMEM((C, HD), jnp.float32),
            pltpu.SemaphoreType.DMA,
        ],
    )
    def k(table_r, idx_r, wgt_r, out_r, idx_v, wgt_v, rows_v, out_v, sem):
        wid = lax.axis_index("s") * NC + lax.axis_index("c")
        b = wid // NH
        h = wid % NH

        def chunk_body(j, carry):
            qb = j * C
            row0 = wid * (nq * NS_PER // 128) + j * GR
            pltpu.sync_copy(idx_r.at[pl.ds(row0, GR)], idx_v)
            pltpu.sync_copy(wgt_r.at[wid, pl.ds(qb, C)], wgt_v)
            descs = [
                pltpu.async_copy(table_r.at[idx_v.at[g]],
                                 rows_v.at[pl.ds(g * 128, 128)], sem)
                for g in range(GR)
            ]
            for d in descs:
                d.wait()

            def item_body(i, c2):
                w16 = wgt_v[i]
                acc0 = jnp.zeros((LANES,), jnp.float32)
                acc1 = jnp.zeros((LANES,), jnp.float32)
                for s in range(NS_PER):
                    ws = _bcast_lane(w16, s)
                    r = i * NS_PER + s
                    acc0 = acc0 + ws * rows_v[r, pl.ds(0, LANES)]
                    acc1 = acc1 + ws * rows_v[r, pl.ds(LANES, LANES)]
                out_v[i, pl.ds(0, LANES)] = acc0
                out_v[i, pl.ds(LANES, LANES)] = acc1
                return c2

            lax.fori_loop(0, C, item_body, 0)
            pltpu.sync_copy(out_v, out_r.at[b, h, pl.ds(qb, C)])
            return carry

        lax.fori_loop(0, NCH, chunk_body, 0)

    return k(table, idx2d, wgt3)


def _out_body(ho_ref, w_ref, b_ref, o_ref, *, TQ):
    acc = jnp.zeros((TQ, EMBED), jnp.float32)
    for h in range(NH):
        acc = acc + jnp.dot(ho_ref[0, h], w_ref[h],
                            preferred_element_type=jnp.float32)
    o_ref[0] = acc + b_ref[...]


def _recombine(headout, wout2, bout, TQ):
    bs, _, nq, _ = headout.shape
    body = functools.partial(_out_body, TQ=TQ)
    return pl.pallas_call(
        body,
        grid=(bs, nq // TQ),
        in_specs=[
            pl.BlockSpec((1, NH, TQ, HD), lambda b, t: (b, 0, t, 0)),
            pl.BlockSpec((NH, HD, EMBED), lambda b, t: (0, 0, 0)),
            pl.BlockSpec((1, EMBED), lambda b, t: (0, 0)),
        ],
        out_specs=pl.BlockSpec((1, TQ, EMBED), lambda b, t: (b, t, 0)),
        out_shape=jax.ShapeDtypeStruct((bs, nq, EMBED), jnp.float32),
    )(headout, wout2, bout)


def kernel(query, reference_points, value, Wv, bv, Wo, bo, Wa, ba, Wout,
           bout, spatial_shape):
    bs, nq, _ = query.shape
    HW = value.shape[1]
    Hs = int(math.isqrt(HW))
    Ws = HW // Hs

    v = _matmul_bias(value.reshape(bs * HW, EMBED), Wv.T,
                     bv.reshape(1, EMBED), blk=1024)
    table = v.reshape(bs * HW * NH, HD)

    Wo4 = Wo.reshape(NH, NP, 2, EMBED)
    wox = Wo4[:, :, 0, :].transpose(0, 2, 1)
    woy = Wo4[:, :, 1, :].transpose(0, 2, 1)
    bo4 = bo.reshape(NH, NP, 2)
    box = bo4[:, :, 0].reshape(NH, 1, NP)
    boy = bo4[:, :, 1].reshape(NH, 1, NP)
    wa = Wa.reshape(NH, NP, EMBED).transpose(0, 2, 1)
    ba3 = ba.reshape(NH, 1, NP)
    idx, wgt = _sampling_prep(query, reference_points, wox, woy, wa,
                              box, boy, ba3, Hs, Ws, TQ=512)

    idx2d = idx.reshape(bs * NH * nq * NS_PER // 128, 128)
    wgt3 = wgt.reshape(bs * NH, nq, NS_PER)
    headout = _sc_sample_combine(table, idx2d, wgt3, bs, nq)

    wout2 = Wout.T.reshape(NH, HD, EMBED)
    return _recombine(headout, wout2, bout.reshape(1, EMBED), TQ=512)


# speedup vs baseline: 8.7066x; 1.7223x over previous
import functools
import math

import jax
import jax.numpy as jnp
from jax import lax
from jax.experimental import pallas as pl
from jax.experimental.pallas import tpu as pltpu
from jax.experimental.pallas import tpu_sc as plsc

EMBED = 256
NH = 8
NP = 4
HD = EMBED // NH
NCORN = 4
NSAMP = NH * NP * NCORN
NC, NS, LANES = 2, 16, 16
NW = NC * NS


def _matmul_bias_body(x_ref, w_ref, b_ref, o_ref):
    o_ref[...] = (
        jnp.dot(x_ref[...], w_ref[...], preferred_element_type=jnp.float32)
        + b_ref[...]
    )


def _matmul_bias(x, w, b, blk):
    m, k = x.shape
    n = w.shape[1]
    return pl.pallas_call(
        _matmul_bias_body,
        grid=(m // blk,),
        in_specs=[
            pl.BlockSpec((blk, k), lambda i: (i, 0)),
            pl.BlockSpec((k, n), lambda i: (0, 0)),
            pl.BlockSpec((1, n), lambda i: (0, 0)),
        ],
        out_specs=pl.BlockSpec((blk, n), lambda i: (i, 0)),
        out_shape=jax.ShapeDtypeStruct((m, n), jnp.float32),
    )(x, w, b)


def _prep_body(q_ref, rp_ref, wox_ref, woy_ref, wa_ref, bo_ref, ba_ref,
               s_ref, idx_ref, wgt_ref, *, Hs, Ws, HW, TQ):
    b = pl.program_id(0)
    q = q_ref[0]
    rp = rp_ref[0]
    rpx = rp[:, 0:1]
    rpy = rp[:, 1:2]
    offx = jnp.dot(q, wox_ref[...], preferred_element_type=jnp.float32) + bo_ref[0:1]
    offy = jnp.dot(q, woy_ref[...], preferred_element_type=jnp.float32) + bo_ref[1:2]
    lg = jnp.dot(q, wa_ref[...], preferred_element_type=jnp.float32) + ba_ref[...]
    m = jnp.max(lg, axis=1, keepdims=True)
    e = jnp.exp(lg - m)
    aw = e / jnp.dot(e, s_ref[...], preferred_element_type=jnp.float32)
    x = rpx * Ws + offx - 0.5
    y = rpy * Hs + offy - 0.5
    x0 = jnp.floor(x)
    fx = x - x0
    x1 = x0 + 1.0
    y0 = jnp.floor(y)
    fy = y - y0
    y1 = y0 + 1.0
    xs = jnp.concatenate([x0, x1, x0, x1], axis=1)
    ys = jnp.concatenate([y0, y0, y1, y1], axis=1)
    wxs = jnp.concatenate([1.0 - fx, fx, 1.0 - fx, fx], axis=1)
    wys = jnp.concatenate([1.0 - fy, 1.0 - fy, fy, fy], axis=1)
    aw4 = jnp.concatenate([aw, aw, aw, aw], axis=1)
    valid = (xs >= 0) & (xs <= Ws - 1) & (ys >= 0) & (ys <= Hs - 1)
    wgt_ref[0] = jnp.where(valid, aw4 * wxs * wys, 0.0)
    ixc = jnp.clip(xs, 0, Ws - 1).astype(jnp.int32)
    iyc = jnp.clip(ys, 0, Hs - 1).astype(jnp.int32)
    hlane = (lax.broadcasted_iota(jnp.int32, (TQ, NSAMP), 1) % (NH * NP)) // NP
    idx_ref[0] = b * (HW * NH) + (iyc * Ws + ixc) * NH + hlane


def _sampling_prep(query, rp, wox, woy, wa, bo2, ba2, smat, Hs, Ws, TQ):
    bs, nq, _ = query.shape
    HW = Hs * Ws
    body = functools.partial(_prep_body, Hs=Hs, Ws=Ws, HW=HW, TQ=TQ)
    hp = NH * NP
    return pl.pallas_call(
        body,
        grid=(bs, nq // TQ),
        in_specs=[
            pl.BlockSpec((1, TQ, EMBED), lambda b, t: (b, t, 0)),
            pl.BlockSpec((1, TQ, 2), lambda b, t: (b, t, 0)),
            pl.BlockSpec((EMBED, hp), lambda b, t: (0, 0)),
            pl.BlockSpec((EMBED, hp), lambda b, t: (0, 0)),
            pl.BlockSpec((EMBED, hp), lambda b, t: (0, 0)),
            pl.BlockSpec((2, hp), lambda b, t: (0, 0)),
            pl.BlockSpec((1, hp), lambda b, t: (0, 0)),
            pl.BlockSpec((hp, hp), lambda b, t: (0, 0)),
        ],
        out_specs=[
            pl.BlockSpec((1, TQ, NSAMP), lambda b, t: (b, t, 0)),
            pl.BlockSpec((1, TQ, NSAMP), lambda b, t: (b, t, 0)),
        ],
        out_shape=[
            jax.ShapeDtypeStruct((bs, nq, NSAMP), jnp.int32),
            jax.ShapeDtypeStruct((bs, nq, NSAMP), jnp.float32),
        ],
    )(query, rp, wox, woy, wa, bo2, ba2, smat)


def _bcast_lane(vec, s):
    return jnp.broadcast_to(lax.slice(vec, (s,), (s + 1,)), (LANES,))


def _sc_sample_combine(table, idx2, wgt2, nrows):
    BQ = idx2.shape[0]
    QW = BQ // NW
    C = 16
    NCH = QW // C
    GR = C * NSAMP // 128
    mesh = plsc.VectorSubcoreMesh(core_axis_name="c", subcore_axis_name="s")

    @functools.partial(
        pl.kernel,
        mesh=mesh,
        compiler_params=pltpu.CompilerParams(use_tc_tiling_on_sc=False),
        out_type=jax.ShapeDtypeStruct((BQ, EMBED), jnp.float32),
        scratch_types=[
            pltpu.VMEM((C, NSAMP), jnp.int32),
            pltpu.VMEM((C, NSAMP), jnp.float32),
            pltpu.VMEM((C * NSAMP, HD), jnp.float32),
            pltpu.VMEM((C, EMBED), jnp.float32),
            pltpu.SemaphoreType.DMA,
        ],
    )
    def k(table_r, idx_r, wgt_r, out_r, idx_v, wgt_v, rows_v, out_v, sem):
        wid = lax.axis_index("s") * NC + lax.axis_index("c")

        def chunk_body(j, carry):
            q0 = wid * QW + j * C
            pltpu.sync_copy(idx_r.at[pl.ds(q0, C)], idx_v)
            pltpu.sync_copy(wgt_r.at[pl.ds(q0, C)], wgt_v)
            descs = [
                pltpu.async_copy(table_r.at[idx_v.at[g]],
                                 rows_v.at[pl.ds(g * 128, 128)], sem)
                for g in range(GR)
            ]
            for d in descs:
                d.wait()

            def item_body(i, c2):
                wvecs = [wgt_v[i, pl.ds(g * LANES, LANES)]
                         for g in range(NSAMP // LANES)]
                for h in range(NH):
                    acc0 = jnp.zeros((LANES,), jnp.float32)
                    acc1 = jnp.zeros((LANES,), jnp.float32)
                    for c in range(NCORN):
                        for p in range(NP):
                            s = c * (NH * NP) + h * NP + p
                            ws = _bcast_lane(wvecs[s // LANES], s % LANES)
                            r = i * NSAMP + s
                            acc0 = acc0 + ws * rows_v[r, pl.ds(0, LANES)]
                            acc1 = acc1 + ws * rows_v[r, pl.ds(LANES, LANES)]
                    out_v[i, pl.ds(h * HD, LANES)] = acc0
                    out_v[i, pl.ds(h * HD + LANES, LANES)] = acc1
                return c2

            lax.fori_loop(0, C, item_body, 0)
            pltpu.sync_copy(out_v, out_r.at[pl.ds(q0, C)])
            return carry

        lax.fori_loop(0, NCH, chunk_body, 0)

    return k(table, idx2, wgt2)


def kernel(query, reference_points, value, Wv, bv, Wo, bo, Wa, ba, Wout,
           bout, spatial_shape):
    bs, nq, _ = query.shape
    HW = value.shape[1]
    Hs = int(math.isqrt(HW))
    Ws = HW // Hs

    v = _matmul_bias(value.reshape(bs * HW, EMBED), Wv.T,
                     bv.reshape(1, EMBED), blk=1024)
    table = v.reshape(bs * HW * NH, HD)

    hp = NH * NP
    Wo4 = Wo.reshape(hp, 2, EMBED)
    wox = Wo4[:, 0, :].T
    woy = Wo4[:, 1, :].T
    bo2 = bo.reshape(hp, 2).T
    wa = Wa.T
    ba2 = ba.reshape(1, hp)
    gid = jnp.arange(hp, dtype=jnp.int32) // NP
    smat = (gid[:, None] == gid[None, :]).astype(jnp.float32)
    idx, wgt = _sampling_prep(query, reference_points, wox, woy, wa,
                              bo2, ba2, smat, Hs, Ws, TQ=512)

    headout = _sc_sample_combine(table, idx.reshape(bs * nq, NSAMP),
                                 wgt.reshape(bs * nq, NSAMP),
                                 bs * HW * NH)

    out = _matmul_bias(headout, Wout.T, bout.reshape(1, EMBED), blk=1024)
    return out.reshape(bs, nq, EMBED)
